# baseline (speedup 1.0000x reference)
import functools

import jax
import jax.numpy as jnp
from jax import lax
from jax.experimental import pallas as pl
from jax.experimental.pallas import tpu as pltpu
from jax.experimental.pallas import tpu_sc as plsc

T = 2048
H = 1024
E = 64
K = 2
I = 64
A = T * K
BM = 64
BUF = 8192
NB = BUF // BM
NW = 32
TB = 256


def _gating_body(x_ref, wg_ref, i0_ref, i1_ref, w0_ref, w1_ref):
    xb = x_ref[...]
    logits = jnp.dot(xb, wg_ref[...], preferred_element_type=jnp.float32)
    m = jnp.max(logits, axis=-1, keepdims=True)
    ex = jnp.exp(logits - m)
    raw = ex / jnp.sum(ex, axis=-1, keepdims=True)
    lane = lax.broadcasted_iota(jnp.int32, raw.shape, 1)
    p1 = jnp.max(raw, axis=-1, keepdims=True)
    a1 = jnp.min(jnp.where(raw == p1, lane, E), axis=-1, keepdims=True)
    masked = jnp.where(lane == a1, -1.0, raw)
    p2 = jnp.max(masked, axis=-1, keepdims=True)
    a2 = jnp.min(jnp.where(masked == p2, lane, E), axis=-1, keepdims=True)
    e2 = jnp.exp(p2 - p1)
    w0 = 1.0 / (1.0 + e2)
    i0_ref[...] = a1.astype(jnp.int32)
    i1_ref[...] = a2.astype(jnp.int32)
    w0_ref[...] = jnp.broadcast_to(w0, (TB, 16))
    w1_ref[...] = jnp.broadcast_to(1.0 - w0, (TB, 16))


def _gating(xf, w_gate):
    col_i = jax.ShapeDtypeStruct((T, 1), jnp.int32)
    spl_f = jax.ShapeDtypeStruct((T, 16), jnp.float32)
    return pl.pallas_call(
        _gating_body,
        grid=(T // TB,),
        in_specs=[
            pl.BlockSpec((TB, H), lambda i: (i, 0)),
            pl.BlockSpec((H, E), lambda i: (0, 0)),
        ],
        out_specs=[
            pl.BlockSpec((TB, 1), lambda i: (i, 0)),
            pl.BlockSpec((TB, 1), lambda i: (i, 0)),
            pl.BlockSpec((TB, 16), lambda i: (i, 0)),
            pl.BlockSpec((TB, 16), lambda i: (i, 0)),
        ],
        out_shape=[col_i, col_i, spl_f, spl_f],
    )(xf, w_gate)


def _routing_body(idx_ref, pos_ref, be_ref, oh_ref):
    idx = idx_ref[...]
    lane = lax.broadcasted_iota(jnp.int32, (A, E), 1)
    oh = (idx == lane).astype(jnp.float32)
    oh_ref[...] = oh
    counts = jnp.sum(oh, axis=0, keepdims=True)
    padded = jnp.ceil(counts / BM) * BM
    li = lax.broadcasted_iota(jnp.int32, (E, E), 0)
    lj = lax.broadcasted_iota(jnp.int32, (E, E), 1)
    ustrict = (li < lj).astype(jnp.float32)
    offs = jnp.dot(padded, ustrict, preferred_element_type=jnp.float32)

    bstart = (lax.broadcasted_iota(jnp.int32, (NB, E), 0) * BM).astype(jnp.float32)
    lane_e = lax.broadcasted_iota(jnp.int32, (NB, E), 1)
    sel = (bstart >= offs) & (bstart < offs + padded)
    be = jnp.sum(jnp.where(sel, lane_e, 0), axis=1, keepdims=True)
    hit = jnp.sum(sel.astype(jnp.int32), axis=1, keepdims=True) > 0
    be_ref[...] = jnp.where(hit, be, -1).astype(jnp.int32)

    ci = lax.broadcasted_iota(jnp.int32, (TB, TB), 0)
    cj = lax.broadcasted_iota(jnp.int32, (TB, TB), 1)
    lstrict = (cj < ci).astype(jnp.float32)

    def chunk(c, run):
        ohc = oh_ref[pl.ds(c * TB, TB), :]
        prior = jnp.dot(lstrict, ohc, preferred_element_type=jnp.float32)
        posf = jnp.sum(ohc * (prior + run + offs), axis=1, keepdims=True)
        pos_ref[pl.ds(c * TB, TB), :] = posf.astype(jnp.int32)
        return run + jnp.sum(ohc, axis=0, keepdims=True)

    lax.fori_loop(0, A // TB, chunk, jnp.zeros((1, E), jnp.float32))


def _routing(idx_all):
    return pl.pallas_call(
        _routing_body,
        out_shape=[
            jax.ShapeDtypeStruct((A, 1), jnp.int32),
            jax.ShapeDtypeStruct((NB, 1), jnp.int32),
        ],
        scratch_shapes=[pltpu.VMEM((A, E), jnp.float32)],
    )(idx_all)


def _dispatch_body(x_hbm, pos_hbm, xs_hbm, rowbuf, posbuf, sem):
    c = lax.axis_index("c")
    s = lax.axis_index("s")
    wid = s * 2 + c

    rows_per = A // NW
    chunk = 64
    for j in range(rows_per // chunk):
        a0 = wid * rows_per + j * chunk
        t0 = lax.rem(a0, T)
        pltpu.sync_copy(x_hbm.at[pl.ds(t0, chunk)], rowbuf)
        pltpu.sync_copy(pos_hbm.at[pl.ds(a0, chunk)], posbuf)
        pltpu.async_copy(rowbuf, xs_hbm.at[posbuf], sem).wait()


def _dispatch(xf, pos_flat):
    mesh = plsc.VectorSubcoreMesh(core_axis_name="c", subcore_axis_name="s")
    fn = pl.kernel(
        _dispatch_body,
        out_type=jax.ShapeDtypeStruct((BUF, H), jnp.float32),
        mesh=mesh,
        scratch_types=[
            pltpu.VMEM((64, H), jnp.float32),
            pltpu.VMEM((64,), jnp.int32),
            pltpu.SemaphoreType.DMA,
        ],
    )
    return fn(xf, pos_flat)


def _mlp_body(be_ref, xs_ref, w1_ref, b1_ref, w2_ref, b2_ref, ys_ref):
    b = pl.program_id(0)
    e = be_ref[b]

    @pl.when(e >= 0)
    def _():
        xb = xs_ref[...]
        h = jnp.dot(xb, w1_ref[0], preferred_element_type=jnp.float32)
        h = h + b1_ref[0]
        g = 0.5 * h * (1.0 + lax.erf(h * 0.7071067811865476))
        y = jnp.dot(g, w2_ref[0], preferred_element_type=jnp.float32)
        ys_ref[...] = y + b2_ref[0]


def _mlp(be, xs, W1, b1r, W2, b2r):
    def xin(b, be_ref):
        return (jnp.where(be_ref[b] >= 0, b, 0), 0)

    def ein(b, be_ref):
        return (jnp.maximum(be_ref[b], 0), 0, 0)

    def yout(b, be_ref):
        return (jnp.where(be_ref[b] >= 0, b, NB - 1), 0)

    grid_spec = pltpu.PrefetchScalarGridSpec(
        num_scalar_prefetch=1,
        grid=(NB,),
        in_specs=[
            pl.BlockSpec((BM, H), xin),
            pl.BlockSpec((1, H, I), ein),
            pl.BlockSpec((1, 1, I), ein),
            pl.BlockSpec((1, I, H), ein),
            pl.BlockSpec((1, 1, H), ein),
        ],
        out_specs=pl.BlockSpec((BM, H), yout),
    )
    return pl.pallas_call(
        _mlp_body,
        grid_spec=grid_spec,
        out_shape=jax.ShapeDtypeStruct((BUF, H), jnp.float32),
    )(be, xs, W1, b1r, W2, b2r)


def _combine_body(ys_hbm, pos_hbm, w0_hbm, w1_hbm, out_hbm,
                  i0buf, i1buf, w0buf, w1buf, bufa, bufb, sema, semb):
    c = lax.axis_index("c")
    s = lax.axis_index("s")
    wid = s * 2 + c

    rows_per = T // NW
    chunk = 32
    for j in range(rows_per // chunk):
        t0 = wid * rows_per + j * chunk
        pltpu.sync_copy(pos_hbm.at[pl.ds(t0, chunk)], i0buf)
        pltpu.sync_copy(pos_hbm.at[pl.ds(T + t0, chunk)], i1buf)
        pltpu.sync_copy(w0_hbm.at[pl.ds(t0, chunk)], w0buf)
        pltpu.sync_copy(w1_hbm.at[pl.ds(t0, chunk)], w1buf)
        cpa = pltpu.async_copy(ys_hbm.at[i0buf], bufa, sema)
        cpb = pltpu.async_copy(ys_hbm.at[i1buf], bufb, semb)
        cpa.wait()
        cpb.wait()

        def row(r, carry):
            wa = w0buf[r, :]
            wb = w1buf[r, :]

            def lanes(l, carry2):
                sl = pl.ds(l * 16, 16)
                bufa[r, sl] = bufa[r, sl] * wa + bufb[r, sl] * wb
                return carry2
            return lax.fori_loop(0, H // 16, lanes, carry)

        lax.fori_loop(0, chunk, row, 0)
        pltpu.sync_copy(bufa, out_hbm.at[pl.ds(t0, chunk)])


def _combine(ys, pos_flat, w0_flat, w1_flat):
    mesh = plsc.VectorSubcoreMesh(core_axis_name="c", subcore_axis_name="s")
    fn = pl.kernel(
        _combine_body,
        out_type=jax.ShapeDtypeStruct((T, H), jnp.float32),
        mesh=mesh,
        scratch_types=[
            pltpu.VMEM((32,), jnp.int32),
            pltpu.VMEM((32,), jnp.int32),
            pltpu.VMEM((32, 16), jnp.float32),
            pltpu.VMEM((32, 16), jnp.float32),
            pltpu.VMEM((32, H), jnp.float32),
            pltpu.VMEM((32, H), jnp.float32),
            pltpu.SemaphoreType.DMA,
            pltpu.SemaphoreType.DMA,
        ],
    )
    return fn(ys, pos_flat, w0_flat, w1_flat)


def kernel(x, w_gate, W1, b1, W2, b2):
    xf = x.reshape(T, H)
    i0, i1, w0, w1 = _gating(xf, w_gate)
    idx_all = jnp.concatenate([i0, i1], axis=0)
    pos, be = _routing(idx_all)
    pos_flat = pos.reshape(A)
    xs = _dispatch(xf, pos_flat)
    ys = _mlp(be.reshape(NB), xs,
              W1, b1.reshape(E, 1, I), W2, b2.reshape(E, 1, H))
    out = _combine(ys, pos_flat, w0, w1)
    return out.reshape(x.shape)


# speedup vs baseline: 1.0991x; 1.0991x over previous
import functools

import jax
import jax.numpy as jnp
from jax import lax
from jax.experimental import pallas as pl
from jax.experimental.pallas import tpu as pltpu
from jax.experimental.pallas import tpu_sc as plsc

T = 2048
H = 1024
E = 64
K = 2
I = 64
A = T * K
BM = 64
BUF = 8192
NB = BUF // BM
NW = 32
TB = 256


def _gating_body(x_ref, wg_ref, i0_ref, i1_ref, w0_ref, w1_ref):
    xb = x_ref[...]
    logits = jnp.dot(xb, wg_ref[...], preferred_element_type=jnp.float32)
    m = jnp.max(logits, axis=-1, keepdims=True)
    ex = jnp.exp(logits - m)
    raw = ex / jnp.sum(ex, axis=-1, keepdims=True)
    lane = lax.broadcasted_iota(jnp.int32, raw.shape, 1)
    p1 = jnp.max(raw, axis=-1, keepdims=True)
    a1 = jnp.min(jnp.where(raw == p1, lane, E), axis=-1, keepdims=True)
    masked = jnp.where(lane == a1, -1.0, raw)
    p2 = jnp.max(masked, axis=-1, keepdims=True)
    a2 = jnp.min(jnp.where(masked == p2, lane, E), axis=-1, keepdims=True)
    e2 = jnp.exp(p2 - p1)
    w0 = 1.0 / (1.0 + e2)
    i0_ref[...] = a1.astype(jnp.int32)
    i1_ref[...] = a2.astype(jnp.int32)
    w0_ref[...] = jnp.broadcast_to(w0, (TB, 16))
    w1_ref[...] = jnp.broadcast_to(1.0 - w0, (TB, 16))


def _gating(xf, w_gate):
    col_i = jax.ShapeDtypeStruct((T, 1), jnp.int32)
    spl_f = jax.ShapeDtypeStruct((T, 16), jnp.float32)
    return pl.pallas_call(
        _gating_body,
        grid=(T // TB,),
        in_specs=[
            pl.BlockSpec((TB, H), lambda i: (i, 0)),
            pl.BlockSpec((H, E), lambda i: (0, 0)),
        ],
        out_specs=[
            pl.BlockSpec((TB, 1), lambda i: (i, 0)),
            pl.BlockSpec((TB, 1), lambda i: (i, 0)),
            pl.BlockSpec((TB, 16), lambda i: (i, 0)),
            pl.BlockSpec((TB, 16), lambda i: (i, 0)),
        ],
        out_shape=[col_i, col_i, spl_f, spl_f],
    )(xf, w_gate)


def _routing_body(idx_ref, pos_ref, be_ref):
    lane = lax.broadcasted_iota(jnp.int32, (TB, E), 1)
    nch = A // TB
    ohs = []
    run = jnp.zeros((1, E), jnp.float32)
    coffs = []
    for c in range(nch):
        ohc = (idx_ref[pl.ds(c * TB, TB), :] == lane).astype(jnp.float32)
        ohs.append(ohc)
        coffs.append(run)
        run = run + jnp.sum(ohc, axis=0, keepdims=True)
    counts = run
    padded = jnp.ceil(counts / BM) * BM
    li = lax.broadcasted_iota(jnp.int32, (E, E), 0)
    lj = lax.broadcasted_iota(jnp.int32, (E, E), 1)
    ustrict = (li < lj).astype(jnp.float32)
    offs = jnp.dot(padded, ustrict, preferred_element_type=jnp.float32)

    bstart = (lax.broadcasted_iota(jnp.int32, (NB, E), 0) * BM).astype(jnp.float32)
    lane_e = lax.broadcasted_iota(jnp.int32, (NB, E), 1)
    sel = (bstart >= offs) & (bstart < offs + padded)
    be = jnp.sum(jnp.where(sel, lane_e, 0), axis=1, keepdims=True)
    hit = jnp.sum(sel.astype(jnp.int32), axis=1, keepdims=True) > 0
    be_ref[...] = jnp.where(hit, be, -1).astype(jnp.int32)

    ci = lax.broadcasted_iota(jnp.int32, (TB, TB), 0)
    cj = lax.broadcasted_iota(jnp.int32, (TB, TB), 1)
    lstrict = (cj < ci).astype(jnp.float32)
    for c in range(nch):
        prior = jnp.dot(lstrict, ohs[c], preferred_element_type=jnp.float32)
        posf = jnp.sum(ohs[c] * (prior + coffs[c] + offs), axis=1, keepdims=True)
        pos_ref[pl.ds(c * TB, TB), :] = posf.astype(jnp.int32)


def _routing(idx_all):
    return pl.pallas_call(
        _routing_body,
        out_shape=[
            jax.ShapeDtypeStruct((A, 1), jnp.int32),
            jax.ShapeDtypeStruct((NB, 1), jnp.int32),
        ],
    )(idx_all)


def _dispatch_body(x_hbm, pos_hbm, xs_hbm, rowbuf, posbuf, sem):
    c = lax.axis_index("c")
    s = lax.axis_index("s")
    wid = s * 2 + c

    rows_per = A // NW
    chunk = 64
    for j in range(rows_per // chunk):
        a0 = wid * rows_per + j * chunk
        t0 = lax.rem(a0, T)
        pltpu.sync_copy(x_hbm.at[pl.ds(t0, chunk)], rowbuf)
        pltpu.sync_copy(pos_hbm.at[pl.ds(a0, chunk)], posbuf)
        pltpu.async_copy(rowbuf, xs_hbm.at[posbuf], sem).wait()


def _dispatch(xf, pos_flat):
    mesh = plsc.VectorSubcoreMesh(core_axis_name="c", subcore_axis_name="s")
    fn = pl.kernel(
        _dispatch_body,
        out_type=jax.ShapeDtypeStruct((BUF, H), jnp.float32),
        mesh=mesh,
        scratch_types=[
            pltpu.VMEM((64, H), jnp.float32),
            pltpu.VMEM((64,), jnp.int32),
            pltpu.SemaphoreType.DMA,
        ],
    )
    return fn(xf, pos_flat)


def _mlp_body(be_ref, xs_ref, w1_ref, b1_ref, w2_ref, b2_ref, ys_ref):
    b = pl.program_id(0)
    e = be_ref[b]

    @pl.when(e >= 0)
    def _():
        xb = xs_ref[...]
        h = jnp.dot(xb, w1_ref[0], preferred_element_type=jnp.float32)
        h = h + b1_ref[0]
        g = 0.5 * h * (1.0 + lax.erf(h * 0.7071067811865476))
        y = jnp.dot(g, w2_ref[0], preferred_element_type=jnp.float32)
        ys_ref[...] = y + b2_ref[0]


def _mlp(be, xs, W1, b1r, W2, b2r):
    def xin(b, be_ref):
        return (jnp.where(be_ref[b] >= 0, b, 0), 0)

    def ein(b, be_ref):
        return (jnp.maximum(be_ref[b], 0), 0, 0)

    def yout(b, be_ref):
        return (jnp.where(be_ref[b] >= 0, b, NB - 1), 0)

    grid_spec = pltpu.PrefetchScalarGridSpec(
        num_scalar_prefetch=1,
        grid=(NB,),
        in_specs=[
            pl.BlockSpec((BM, H), xin),
            pl.BlockSpec((1, H, I), ein),
            pl.BlockSpec((1, 1, I), ein),
            pl.BlockSpec((1, I, H), ein),
            pl.BlockSpec((1, 1, H), ein),
        ],
        out_specs=pl.BlockSpec((BM, H), yout),
    )
    return pl.pallas_call(
        _mlp_body,
        grid_spec=grid_spec,
        out_shape=jax.ShapeDtypeStruct((BUF, H), jnp.float32),
    )(be, xs, W1, b1r, W2, b2r)


def _combine_body(ys_hbm, pos_hbm, w0_hbm, w1_hbm, out_hbm,
                  i0buf, i1buf, w0buf, w1buf, bufa, bufb, sema, semb):
    c = lax.axis_index("c")
    s = lax.axis_index("s")
    wid = s * 2 + c

    rows_per = T // NW
    chunk = 32
    for j in range(rows_per // chunk):
        t0 = wid * rows_per + j * chunk
        pltpu.sync_copy(pos_hbm.at[pl.ds(t0, chunk)], i0buf)
        pltpu.sync_copy(pos_hbm.at[pl.ds(T + t0, chunk)], i1buf)
        pltpu.sync_copy(w0_hbm.at[pl.ds(t0, chunk)], w0buf)
        pltpu.sync_copy(w1_hbm.at[pl.ds(t0, chunk)], w1buf)
        cpa = pltpu.async_copy(ys_hbm.at[i0buf], bufa, sema)
        cpb = pltpu.async_copy(ys_hbm.at[i1buf], bufb, semb)
        cpa.wait()
        cpb.wait()

        def row(r, carry):
            wa = w0buf[r, :]
            wb = w1buf[r, :]
            for l in range(H // 16):
                sl = pl.ds(l * 16, 16)
                bufa[r, sl] = bufa[r, sl] * wa + bufb[r, sl] * wb
            return carry

        lax.fori_loop(0, chunk, row, 0)
        pltpu.sync_copy(bufa, out_hbm.at[pl.ds(t0, chunk)])


def _combine(ys, pos_flat, w0_flat, w1_flat):
    mesh = plsc.VectorSubcoreMesh(core_axis_name="c", subcore_axis_name="s")
    fn = pl.kernel(
        _combine_body,
        out_type=jax.ShapeDtypeStruct((T, H), jnp.float32),
        mesh=mesh,
        scratch_types=[
            pltpu.VMEM((32,), jnp.int32),
            pltpu.VMEM((32,), jnp.int32),
            pltpu.VMEM((32, 16), jnp.float32),
            pltpu.VMEM((32, 16), jnp.float32),
            pltpu.VMEM((32, H), jnp.float32),
            pltpu.VMEM((32, H), jnp.float32),
            pltpu.SemaphoreType.DMA,
            pltpu.SemaphoreType.DMA,
        ],
    )
    return fn(ys, pos_flat, w0_flat, w1_flat)


def kernel(x, w_gate, W1, b1, W2, b2):
    xf = x.reshape(T, H)
    i0, i1, w0, w1 = _gating(xf, w_gate)
    idx_all = jnp.concatenate([i0, i1], axis=0)
    pos, be = _routing(idx_all)
    pos_flat = pos.reshape(A)
    xs = _dispatch(xf, pos_flat)
    ys = _mlp(be.reshape(NB), xs,
              W1, b1.reshape(E, 1, I), W2, b2.reshape(E, 1, H))
    out = _combine(ys, pos_flat, w0, w1)
    return out.reshape(x.shape)
